# Initial kernel scaffold; baseline (speedup 1.0000x reference)
import jax
import jax.numpy as jnp
from jax.experimental import pallas as pl


def kernel(x, weight):
    raise NotImplementedError("write your pallas kernel here")


# speedup vs baseline: 3.5439x; 3.5439x over previous
import functools

import jax
import jax.numpy as jnp
from jax import lax
from jax.experimental import pallas as pl
from jax.experimental.pallas import tpu as pltpu
from jax.experimental.pallas import tpu_sc as plsc

_NC = 2
_NS = 16
_NW = _NC * _NS

_CHUNK = 128


@functools.lru_cache(maxsize=None)
def _gather_kernel(N, D, ch):
    n_per_w = N // _NW
    n_ch = n_per_w // ch
    mesh = plsc.VectorSubcoreMesh(core_axis_name="c", subcore_axis_name="s")

    @functools.partial(
        pl.kernel,
        mesh=mesh,
        compiler_params=pltpu.CompilerParams(use_tc_tiling_on_sc=False),
        out_type=jax.ShapeDtypeStruct((N, D), jnp.float32),
        scratch_types=[
            pltpu.VMEM((n_per_w,), jnp.int32),
            pltpu.VMEM((ch, D), jnp.float32),
            pltpu.SemaphoreType.DMA,
        ],
    )
    def k(x_hbm, w_hbm, out_hbm, idx_v, rows_v, sem):
        wid = lax.axis_index("s") * _NC + lax.axis_index("c")
        base = wid * n_per_w
        pltpu.sync_copy(x_hbm.at[pl.ds(base, n_per_w)], idx_v)

        def body(g, carry):
            off = g * ch
            pltpu.async_copy(
                w_hbm.at[idx_v.at[pl.ds(off, ch)]], rows_v, sem
            ).wait()
            pltpu.sync_copy(rows_v, out_hbm.at[pl.ds(base + off, ch)])
            return carry

        lax.fori_loop(0, n_ch, body, 0)

    return k


def kernel(x, weight):
    B, T = x.shape
    V, D = weight.shape
    N = B * T
    out = _gather_kernel(N, D, _CHUNK)(x.reshape(N), weight)
    return out.reshape(B, T, D)


# speedup vs baseline: 4.2154x; 1.1895x over previous
import functools

import jax
import jax.numpy as jnp
from jax import lax
from jax.experimental import pallas as pl
from jax.experimental.pallas import tpu as pltpu
from jax.experimental.pallas import tpu_sc as plsc

_NC = 2
_NS = 16
_NW = _NC * _NS

_CHUNK = 256
_NB = 4


@functools.lru_cache(maxsize=None)
def _gather_kernel(N, D, ch, nb):
    n_per_w = N // _NW
    n_ch = n_per_w // ch
    assert n_ch % nb == 0
    mesh = plsc.VectorSubcoreMesh(core_axis_name="c", subcore_axis_name="s")

    @functools.partial(
        pl.kernel,
        mesh=mesh,
        compiler_params=pltpu.CompilerParams(use_tc_tiling_on_sc=False),
        out_type=jax.ShapeDtypeStruct((N, D), jnp.float32),
        scratch_types=[
            pltpu.VMEM((n_per_w,), jnp.int32),
            pltpu.VMEM((nb, ch, D), jnp.float32),
        ]
        + [pltpu.SemaphoreType.DMA] * (2 * nb),
    )
    def k(x_hbm, w_hbm, out_hbm, idx_v, rows_v, *sems):
        gsem, wsem = sems[:nb], sems[nb:]
        wid = lax.axis_index("s") * _NC + lax.axis_index("c")
        base = wid * n_per_w
        pltpu.sync_copy(x_hbm.at[pl.ds(base, n_per_w)], idx_v)

        def gather(g, b):
            pltpu.async_copy(
                w_hbm.at[idx_v.at[pl.ds(g * ch, ch)]], rows_v.at[b], gsem[b]
            )

        for b in range(nb):
            gather(b, b)

        def body(i, carry):
            g0 = i * nb
            for b in range(nb):
                pltpu.make_async_copy(
                    w_hbm.at[idx_v.at[pl.ds(0, ch)]], rows_v.at[b], gsem[b]
                ).wait()
                pltpu.async_copy(
                    rows_v.at[b], out_hbm.at[pl.ds(base + (g0 + b) * ch, ch)],
                    wsem[b],
                )
            for b in range(nb):
                pltpu.make_async_copy(
                    rows_v.at[b], out_hbm.at[pl.ds(base, ch)], wsem[b]
                ).wait()
                gather(jnp.minimum(g0 + nb + b, n_ch - 1), b)
            return carry

        lax.fori_loop(0, n_ch // nb, body, 0)

        for b in range(nb):
            pltpu.make_async_copy(
                w_hbm.at[idx_v.at[pl.ds(0, ch)]], rows_v.at[b], gsem[b]
            ).wait()

    return k


def kernel(x, weight):
    B, T = x.shape
    V, D = weight.shape
    N = B * T
    out = _gather_kernel(N, D, _CHUNK, _NB)(x.reshape(N), weight)
    return out.reshape(B, T, D)
